# baseline (speedup 1.0000x reference)
import functools

import jax
import jax.numpy as jnp
from jax import lax
from jax.experimental import pallas as pl
from jax.experimental.pallas import tpu as pltpu
from jax.experimental.pallas import tpu_sc as plsc

D = 2048
B = 4
S = 2048

_info = plsc.get_sparse_core_info()
_NC, _NS, _L = _info.num_cores, _info.num_subcores, _info.num_lanes
_NW = _NC * _NS
_S_PER_W = S // _NW
_CB = 8
_NBLK = _S_PER_W // _CB
_NCHUNK = _NBLK * B
_NR = 5
_NP = 2
_DEPTH = 3
_SLICES = _CB * (D // _L)
_UNROLL = 16


def _add_pe(rows_ref, pe_ref):
    def body(i, c):
        r = i >> 3
        c0 = (i & 7) * (_UNROLL * _L)
        for u in range(_UNROLL):
            col = c0 + u * _L
            rows_ref[r, pl.ds(col, _L)] = (
                rows_ref[r, pl.ds(col, _L)] + pe_ref[r, pl.ds(col, _L)]
            )
        return c
    lax.fori_loop(0, _SLICES // _UNROLL, body, 0)


_mesh = plsc.VectorSubcoreMesh(core_axis_name="c", subcore_axis_name="s")


@functools.partial(
    pl.kernel,
    out_type=jax.ShapeDtypeStruct((B * S, D), jnp.float32),
    mesh=_mesh,
    scratch_types=(
        [pltpu.VMEM((B, _S_PER_W), jnp.int32)]
        + [pltpu.VMEM((_CB, D), jnp.float32) for _ in range(_NR + _NP)]
        + [pltpu.SemaphoreType.DMA for _ in range(_NR + _NR + _NP)]
    ),
)
def _emb(table, tok, pe, out, idx_v, *rest):
    rows = rest[:_NR]
    pes = rest[_NR:_NR + _NP]
    sem_g = rest[_NR + _NP:2 * _NR + _NP]
    sem_o = rest[2 * _NR + _NP:3 * _NR + _NP]
    sem_p = rest[3 * _NR + _NP:]

    wid = lax.axis_index("s") * _NC + lax.axis_index("c")
    s0 = wid * _S_PER_W

    for bt in range(B):
        pltpu.sync_copy(tok.at[pl.ds(bt * S + s0, _S_PER_W)], idx_v.at[bt])

    def gather(j):
        p, bt = divmod(j, B)
        rb = j % _NR
        idx = idx_v.at[bt, pl.ds(p * _CB, _CB)]
        return pltpu.async_copy(table.at[idx], rows[rb], sem_g[rb])

    def pe_load(p):
        return pltpu.async_copy(
            pe.at[pl.ds(s0 + p * _CB, _CB)], pes[p % _NP], sem_p[p % _NP])

    pe_h = [pe_load(0), pe_load(1)]
    g_h = [None] * _NR
    o_h = [None] * _NR
    for j in range(_DEPTH):
        g_h[j % _NR] = gather(j)

    for j in range(_NCHUNK):
        p, bt = divmod(j, B)
        rb = j % _NR
        pb = p % _NP
        g_h[rb].wait()
        if bt == 0:
            pe_h[pb].wait()
        _add_pe(rows[rb], pes[pb])
        o_h[rb] = pltpu.async_copy(
            rows[rb], out.at[pl.ds(bt * S + s0 + p * _CB, _CB)], sem_o[rb])
        if bt == B - 1 and p + _NP < _NBLK:
            pe_h[pb] = pe_load(p + _NP)
        jn = j + _DEPTH
        if jn < _NCHUNK:
            rbn = jn % _NR
            if o_h[rbn] is not None:
                o_h[rbn].wait()
                o_h[rbn] = None
            g_h[rbn] = gather(jn)
    for h in o_h:
        if h is not None:
            h.wait()


def kernel(tokens, embedding_weight, pe):
    tok = tokens.reshape(-1).astype(jnp.int32)
    out = _emb(embedding_weight, tok, pe)
    return out.reshape(B, S, D)


# speedup vs baseline: 1.3414x; 1.3414x over previous
import functools

import jax
import jax.numpy as jnp
from jax import lax
from jax.experimental import pallas as pl
from jax.experimental.pallas import tpu as pltpu
from jax.experimental.pallas import tpu_sc as plsc

D = 2048
B = 4
S = 2048

_info = plsc.get_sparse_core_info()
_NC, _NS, _L = _info.num_cores, _info.num_subcores, _info.num_lanes
_NW = _NC * _NS
_S_PER_W = S // _NW
_CB = 8
_NBLK = _S_PER_W // _CB
_NCHUNK = _NBLK * B
_NR = 5
_NP = 2
_DEPTH = 3
_SLICES = _CB * (D // _L)
_UNROLL = 8


def _add_pe(rows_ref, pe_ref):
    def body(i, c):
        r = i >> 4
        c0 = (i & 15) * (_UNROLL * _L)
        for u in range(_UNROLL):
            col = c0 + u * _L
            rows_ref[r, pl.ds(col, _L)] = (
                rows_ref[r, pl.ds(col, _L)] + pe_ref[r, pl.ds(col, _L)]
            )
        return c
    lax.fori_loop(0, _SLICES // _UNROLL, body, 0)


_mesh = plsc.VectorSubcoreMesh(core_axis_name="c", subcore_axis_name="s")


@functools.partial(
    pl.kernel,
    out_type=jax.ShapeDtypeStruct((B * S, D), jnp.float32),
    mesh=_mesh,
    scratch_types=(
        [pltpu.VMEM((B, _S_PER_W), jnp.int32)]
        + [pltpu.VMEM((_CB, D), jnp.float32) for _ in range(_NR + _NP)]
        + [pltpu.SemaphoreType.DMA for _ in range(_NR + _NR + _NP)]
    ),
)
def _emb(table, tok, pe, out, idx_v, *rest):
    rows = rest[:_NR]
    pes = rest[_NR:_NR + _NP]
    sem_g = rest[_NR + _NP:2 * _NR + _NP]
    sem_o = rest[2 * _NR + _NP:3 * _NR + _NP]
    sem_p = rest[3 * _NR + _NP:]

    wid = lax.axis_index("s") * _NC + lax.axis_index("c")
    s0 = wid * _S_PER_W

    for bt in range(B):
        pltpu.sync_copy(tok.at[pl.ds(bt * S + s0, _S_PER_W)], idx_v.at[bt])

    def gather(j):
        p, bt = divmod(j, B)
        rb = j % _NR
        idx = idx_v.at[bt, pl.ds(p * _CB, _CB)]
        return pltpu.async_copy(table.at[idx], rows[rb], sem_g[rb])

    def pe_load(p):
        return pltpu.async_copy(
            pe.at[pl.ds(s0 + p * _CB, _CB)], pes[p % _NP], sem_p[p % _NP])

    pe_h = [pe_load(0), pe_load(1)]
    g_h = [None] * _NR
    o_h = [None] * _NR
    for j in range(_DEPTH):
        g_h[j % _NR] = gather(j)

    for j in range(_NCHUNK):
        p, bt = divmod(j, B)
        rb = j % _NR
        pb = p % _NP
        g_h[rb].wait()
        if bt == 0:
            pe_h[pb].wait()
        _add_pe(rows[rb], pes[pb])
        o_h[rb] = pltpu.async_copy(
            rows[rb], out.at[pl.ds(bt * S + s0 + p * _CB, _CB)], sem_o[rb])
        if bt == B - 1 and p + _NP < _NBLK:
            pe_h[pb] = pe_load(p + _NP)
        jn = j + _DEPTH
        if jn < _NCHUNK:
            rbn = jn % _NR
            if o_h[rbn] is not None:
                o_h[rbn].wait()
                o_h[rbn] = None
            g_h[rbn] = gather(jn)
    for h in o_h:
        if h is not None:
            h.wait()


def kernel(tokens, embedding_weight, pe):
    tok = tokens.reshape(-1).astype(jnp.int32)
    out = _emb(embedding_weight, tok, pe)
    return out.reshape(B, S, D)


# speedup vs baseline: 1.3569x; 1.0115x over previous
import functools

import jax
import jax.numpy as jnp
from jax import lax
from jax.experimental import pallas as pl
from jax.experimental.pallas import tpu as pltpu
from jax.experimental.pallas import tpu_sc as plsc

D = 2048
B = 4
S = 2048

_info = plsc.get_sparse_core_info()
_NC, _NS, _L = _info.num_cores, _info.num_subcores, _info.num_lanes
_NW = _NC * _NS
_S_PER_W = S // _NW
_CB = 8
_NBLK = _S_PER_W // _CB
_NCHUNK = _NBLK * B
_NR = 5
_NP = 2
_DEPTH = 3
_SLICES = _CB * (D // _L)
_UNROLL = 8


def _add_pe(rows_ref, pe_ref):
    def body(i, c):
        r = i >> 4
        c0 = (i & 15) * (_UNROLL * _L)
        for u in range(_UNROLL):
            col = c0 + u * _L
            rows_ref[r, pl.ds(col, _L)] = (
                rows_ref[r, pl.ds(col, _L)] + pe_ref[r, pl.ds(col, _L)]
            )
        return c
    lax.fori_loop(0, _SLICES // _UNROLL, body, 0)


_mesh = plsc.VectorSubcoreMesh(core_axis_name="c", subcore_axis_name="s")


@functools.partial(
    pl.kernel,
    out_type=jax.ShapeDtypeStruct((B, S, D), jnp.float32),
    mesh=_mesh,
    scratch_types=(
        [pltpu.VMEM((B, _S_PER_W), jnp.int32)]
        + [pltpu.VMEM((_CB, D), jnp.float32) for _ in range(_NR + _NP)]
        + [pltpu.SemaphoreType.DMA for _ in range(_NR + _NR + _NP)]
    ),
)
def _emb(table, tok, pe, out, idx_v, *rest):
    rows = rest[:_NR]
    pes = rest[_NR:_NR + _NP]
    sem_g = rest[_NR + _NP:2 * _NR + _NP]
    sem_o = rest[2 * _NR + _NP:3 * _NR + _NP]
    sem_p = rest[3 * _NR + _NP:]

    wid = lax.axis_index("s") * _NC + lax.axis_index("c")
    s0 = wid * _S_PER_W

    idx_h = [
        pltpu.async_copy(tok.at[bt, pl.ds(s0, _S_PER_W)], idx_v.at[bt],
                         sem_p[0])
        for bt in range(B)
    ]
    for h in idx_h:
        h.wait()

    def gather(j):
        p, bt = divmod(j, B)
        rb = j % _NR
        idx = idx_v.at[bt, pl.ds(p * _CB, _CB)]
        return pltpu.async_copy(table.at[idx], rows[rb], sem_g[rb])

    def pe_load(p):
        return pltpu.async_copy(
            pe.at[pl.ds(s0 + p * _CB, _CB)], pes[p % _NP], sem_p[p % _NP])

    pe_h = [pe_load(0), pe_load(1)]
    g_h = [None] * _NR
    o_h = [None] * _NR
    for j in range(_DEPTH):
        g_h[j % _NR] = gather(j)

    for j in range(_NCHUNK):
        p, bt = divmod(j, B)
        rb = j % _NR
        pb = p % _NP
        g_h[rb].wait()
        if bt == 0:
            pe_h[pb].wait()
        _add_pe(rows[rb], pes[pb])
        o_h[rb] = pltpu.async_copy(
            rows[rb], out.at[bt, pl.ds(s0 + p * _CB, _CB)], sem_o[rb])
        if bt == B - 1 and p + _NP < _NBLK:
            pe_h[pb] = pe_load(p + _NP)
        jn = j + _DEPTH
        if jn < _NCHUNK:
            rbn = jn % _NR
            if o_h[rbn] is not None:
                o_h[rbn].wait()
                o_h[rbn] = None
            g_h[rbn] = gather(jn)
    for h in o_h:
        if h is not None:
            h.wait()


def kernel(tokens, embedding_weight, pe):
    return _emb(embedding_weight, tokens.astype(jnp.int32), pe)


# speedup vs baseline: 1.3593x; 1.0018x over previous
import functools

import jax
import jax.numpy as jnp
from jax import lax
from jax.experimental import pallas as pl
from jax.experimental.pallas import tpu as pltpu
from jax.experimental.pallas import tpu_sc as plsc

D = 2048
B = 4
S = 2048

_info = plsc.get_sparse_core_info()
_NC, _NS, _L = _info.num_cores, _info.num_subcores, _info.num_lanes
_NW = _NC * _NS
_S_PER_W = S // _NW
_CB = 8
_NBLK = _S_PER_W // _CB
_NCHUNK = _NBLK * B
_NR = 5
_NP = 2
_DEPTH = 4
_SLICES = _CB * (D // _L)
_UNROLL = 8


def _add_pe(rows_ref, pe_ref):
    def body(i, c):
        r = i >> 4
        c0 = (i & 15) * (_UNROLL * _L)
        for u in range(_UNROLL):
            col = c0 + u * _L
            rows_ref[r, pl.ds(col, _L)] = (
                rows_ref[r, pl.ds(col, _L)] + pe_ref[r, pl.ds(col, _L)]
            )
        return c
    lax.fori_loop(0, _SLICES // _UNROLL, body, 0)


_mesh = plsc.VectorSubcoreMesh(core_axis_name="c", subcore_axis_name="s")


@functools.partial(
    pl.kernel,
    out_type=jax.ShapeDtypeStruct((B, S, D), jnp.float32),
    mesh=_mesh,
    scratch_types=(
        [pltpu.VMEM((B, _S_PER_W), jnp.int32)]
        + [pltpu.VMEM((_CB, D), jnp.float32) for _ in range(_NR + _NP)]
        + [pltpu.SemaphoreType.DMA for _ in range(_NR + _NR + _NP)]
    ),
)
def _emb(table, tok, pe, out, idx_v, *rest):
    rows = rest[:_NR]
    pes = rest[_NR:_NR + _NP]
    sem_g = rest[_NR + _NP:2 * _NR + _NP]
    sem_o = rest[2 * _NR + _NP:3 * _NR + _NP]
    sem_p = rest[3 * _NR + _NP:]

    wid = lax.axis_index("s") * _NC + lax.axis_index("c")
    s0 = wid * _S_PER_W

    idx_h = [
        pltpu.async_copy(tok.at[bt, pl.ds(s0, _S_PER_W)], idx_v.at[bt],
                         sem_p[0])
        for bt in range(B)
    ]
    for h in idx_h:
        h.wait()

    def gather(j):
        p, bt = divmod(j, B)
        rb = j % _NR
        idx = idx_v.at[bt, pl.ds(p * _CB, _CB)]
        return pltpu.async_copy(table.at[idx], rows[rb], sem_g[rb])

    def pe_load(p):
        return pltpu.async_copy(
            pe.at[pl.ds(s0 + p * _CB, _CB)], pes[p % _NP], sem_p[p % _NP])

    pe_h = [pe_load(0), pe_load(1)]
    g_h = [None] * _NR
    o_h = [None] * _NR
    for j in range(_DEPTH):
        g_h[j % _NR] = gather(j)

    for j in range(_NCHUNK):
        p, bt = divmod(j, B)
        rb = j % _NR
        pb = p % _NP
        g_h[rb].wait()
        if bt == 0:
            pe_h[pb].wait()
        _add_pe(rows[rb], pes[pb])
        o_h[rb] = pltpu.async_copy(
            rows[rb], out.at[bt, pl.ds(s0 + p * _CB, _CB)], sem_o[rb])
        if bt == B - 1 and p + _NP < _NBLK:
            pe_h[pb] = pe_load(p + _NP)
        jn = j + _DEPTH
        if jn < _NCHUNK:
            rbn = jn % _NR
            if o_h[rbn] is not None:
                o_h[rbn].wait()
                o_h[rbn] = None
            g_h[rbn] = gather(jn)
    for h in o_h:
        if h is not None:
            h.wait()


def kernel(tokens, embedding_weight, pe):
    return _emb(embedding_weight, tokens.astype(jnp.int32), pe)


# speedup vs baseline: 1.3650x; 1.0042x over previous
import functools

import jax
import jax.numpy as jnp
from jax import lax
from jax.experimental import pallas as pl
from jax.experimental.pallas import tpu as pltpu
from jax.experimental.pallas import tpu_sc as plsc

D = 2048
B = 4
S = 2048

_info = plsc.get_sparse_core_info()
_NC, _NS, _L = _info.num_cores, _info.num_subcores, _info.num_lanes
_NW = _NC * _NS
_S_PER_W = S // _NW
_CB = 8
_NBLK = _S_PER_W // _CB
_NCHUNK = _NBLK * B
_NR = 5
_NP = 2
_DEPTH = 4
_SLICES = _CB * (D // _L)
_UNROLL = 8


def _add_pe(rows_ref, pe_ref, po, n):
    def body(i, c):
        r = i >> 4
        c0 = (i & 15) * (_UNROLL * _L)
        for u in range(_UNROLL):
            col = c0 + u * _L
            rows_ref[r, pl.ds(col, _L)] = (
                rows_ref[r, pl.ds(col, _L)] + pe_ref[po + r, pl.ds(col, _L)]
            )
        return c
    lax.fori_loop(0, n * (D // _L) // _UNROLL, body, 0)


_mesh = plsc.VectorSubcoreMesh(core_axis_name="c", subcore_axis_name="s")


@functools.partial(
    pl.kernel,
    out_type=jax.ShapeDtypeStruct((B, S, D), jnp.float32),
    mesh=_mesh,
    scratch_types=(
        [pltpu.VMEM((B, _S_PER_W), jnp.int32)]
        + [pltpu.VMEM((_CB, D), jnp.float32) for _ in range(_NR + _NP)]
        + [pltpu.SemaphoreType.DMA for _ in range(_NR + _NR + _NP)]
    ),
)
def _emb(table, tok, pe, out, idx_v, *rest):
    rows = rest[:_NR]
    pes = rest[_NR:_NR + _NP]
    sem_g = rest[_NR + _NP:2 * _NR + _NP]
    sem_o = rest[2 * _NR + _NP:3 * _NR + _NP]
    sem_p = rest[3 * _NR + _NP:]

    wid = lax.axis_index("s") * _NC + lax.axis_index("c")
    s0 = wid * _S_PER_W

    idx_h = [
        pltpu.async_copy(tok.at[bt, pl.ds(s0, _S_PER_W)], idx_v.at[bt],
                         sem_p[0])
        for bt in range(B)
    ]
    _idx_waited = set()

    def gather(j):
        p, bt = divmod(j, B)
        rb = j % _NR
        if bt not in _idx_waited:
            idx_h[bt].wait()
            _idx_waited.add(bt)
        idx = idx_v.at[bt, pl.ds(p * _CB, _CB)]
        return pltpu.async_copy(table.at[idx], rows[rb], sem_g[rb])

    def pe_load(p):
        return pltpu.async_copy(
            pe.at[pl.ds(s0 + p * _CB, _CB)], pes[p % _NP], sem_p[p % _NP])

    pe_h = [pe_load(0), None]
    g_h = [None] * _NR
    o_h = [None] * _NR
    for j in range(_DEPTH):
        g_h[j % _NR] = gather(j)
    pe_h[1] = pe_load(1)

    for j in range(_NCHUNK):
        p, bt = divmod(j, B)
        rb = j % _NR
        pb = p % _NP
        g_h[rb].wait()
        if bt == 0:
            pe_h[pb].wait()
        _add_pe(rows[rb], pes[pb], 0, _CB)
        o_h[rb] = pltpu.async_copy(
            rows[rb], out.at[bt, pl.ds(s0 + p * _CB, _CB)], sem_o[rb])
        if bt == B - 1 and p + _NP < _NBLK:
            pe_h[pb] = pe_load(p + _NP)
        jn = j + _DEPTH
        if jn < _NCHUNK:
            rbn = jn % _NR
            if o_h[rbn] is not None:
                o_h[rbn].wait()
                o_h[rbn] = None
            g_h[rbn] = gather(jn)
    for h in o_h:
        if h is not None:
            h.wait()


def kernel(tokens, embedding_weight, pe):
    return _emb(embedding_weight, tokens.astype(jnp.int32), pe)
